# baseline (speedup 1.0000x reference)
import functools

import jax
import jax.numpy as jnp
from jax import lax
from jax.experimental import pallas as pl
from jax.experimental.pallas import tpu as pltpu
from jax.experimental.pallas import tpu_sc as plsc

NC = 2
NS = 16
NW = NC * NS

_mesh = functools.partial(
    plsc.VectorSubcoreMesh,
    core_axis_name="c", subcore_axis_name="s", num_cores=NC, num_subcores=NS,
)


def _pack16(v):
    half = v.shape[1] // 2
    bits = jax.lax.bitcast_convert_type(v, jnp.uint32) + jnp.uint32(0x8000)
    lo = bits[:, :half] & jnp.uint32(0xFFFF0000)
    hi = bits[:, half:] >> jnp.uint32(16)
    return jax.lax.bitcast_convert_type(lo | hi, jnp.int32)


def _unpack16(w):
    u = jax.lax.bitcast_convert_type(w, jnp.uint32)
    lo = jax.lax.bitcast_convert_type(u & jnp.uint32(0xFFFF0000), jnp.float32)
    hi = jax.lax.bitcast_convert_type(u << jnp.uint32(16), jnp.float32)
    return lo, hi


def _packx_body(x_ref, o_ref):
    o_ref[...] = _pack16(x_ref[...])


def _packx(x, bn=2000):
    n, d = x.shape
    return pl.pallas_call(
        _packx_body,
        grid=(n // bn,),
        in_specs=[pl.BlockSpec((bn, d), lambda i: (i, 0))],
        out_specs=pl.BlockSpec((bn, d // 2), lambda i: (i, 0)),
        out_shape=jax.ShapeDtypeStruct((n, d // 2), jnp.int32),
    )(x)


def _make_gather(e, n, d, c=40):
    per_w = e // NW
    n_chunks = per_w // c
    assert n_chunks % 2 == 1 and n_chunks >= 3
    n_pairs = n_chunks // 2
    w = d // 2

    @functools.partial(
        pl.kernel,
        out_type=(
            jax.ShapeDtypeStruct((e, w), jnp.int32),
            jax.ShapeDtypeStruct((e, w), jnp.int32),
        ),
        mesh=_mesh(),
        scratch_types=[
            pltpu.VMEM((per_w,), jnp.int32),
            pltpu.VMEM((per_w,), jnp.int32),
            pltpu.VMEM((2, c, w), jnp.int32),
            pltpu.VMEM((2, c, w), jnp.int32),
            ([pltpu.SemaphoreType.DMA] * 2),
            ([pltpu.SemaphoreType.DMA] * 2),
            ([pltpu.SemaphoreType.DMA] * 2),
            ([pltpu.SemaphoreType.DMA] * 2),
        ],
    )
    def k(p_hbm, q_hbm, src_hbm, dst_hbm, gp_hbm, gq_hbm,
          idx_s, idx_d, bufp, bufq, sgp, swp, sgq, swq):
        wid = lax.axis_index("s") * NC + lax.axis_index("c")
        base = wid * per_w
        pltpu.sync_copy(src_hbm.at[pl.ds(base, per_w)], idx_s)
        pltpu.sync_copy(dst_hbm.at[pl.ds(base, per_w)], idx_d)

        def g(s, j):
            pltpu.async_copy(
                p_hbm.at[idx_s.at[pl.ds(j * c, c)]], bufp.at[s], sgp[s])
            pltpu.async_copy(
                q_hbm.at[idx_d.at[pl.ds(j * c, c)]], bufq.at[s], sgq[s])

        def wg(s):
            pltpu.make_async_copy(
                p_hbm.at[idx_s.at[pl.ds(0, c)]], bufp.at[s], sgp[s]).wait()
            pltpu.make_async_copy(
                q_hbm.at[idx_d.at[pl.ds(0, c)]], bufq.at[s], sgq[s]).wait()

        def w(s, j):
            pltpu.async_copy(
                bufp.at[s], gp_hbm.at[pl.ds(base + j * c, c)], swp[s])
            pltpu.async_copy(
                bufq.at[s], gq_hbm.at[pl.ds(base + j * c, c)], swq[s])

        def ww(s):
            pltpu.make_async_copy(
                bufp.at[s], gp_hbm.at[pl.ds(base, c)], swp[s]).wait()
            pltpu.make_async_copy(
                bufq.at[s], gq_hbm.at[pl.ds(base, c)], swq[s]).wait()

        g(0, 0)

        def body(i, carry):
            j0 = 2 * i
            @pl.when(i > 0)
            def _():
                ww(1)

            g(1, j0 + 1)
            wg(0)
            w(0, j0)
            ww(0)
            g(0, j0 + 2)
            wg(1)
            w(1, j0 + 1)
            return carry

        lax.fori_loop(0, n_pairs, body, 0)
        wg(0)
        w(0, n_chunks - 1)
        ww(0)
        ww(1)

    return k


def _edge_body(ea_ref, xs_ref, xd_ref, w1a_ref, b1_ref, w1s_ref, w1d_ref,
               w2_ref, b2_ref, lo_ref, hi_ref):
    ea = ea_ref[...]
    dn = w2_ref.shape[1]
    qw = w1s_ref.shape[0] // 2
    pre = (jnp.dot(ea.astype(jnp.bfloat16), w1a_ref[...],
                   preferred_element_type=jnp.float32)
           + b1_ref[...])
    for ref, wref in ((xs_ref, w1s_ref), (xd_ref, w1d_ref)):
        vlo, vhi = _unpack16(ref[...])
        pre = pre + jnp.dot(
            vlo.astype(jnp.bfloat16), wref[:qw, :],
            preferred_element_type=jnp.float32)
        pre = pre + jnp.dot(
            vhi.astype(jnp.bfloat16), wref[qw:, :],
            preferred_element_type=jnp.float32)
    h = jnp.tanh(pre).astype(jnp.bfloat16)
    ne = jnp.tanh(
        jnp.dot(h, w2_ref[...], preferred_element_type=jnp.float32)
        + b2_ref[...])
    half = dn // 2
    lo_ref[...] = ne[:, :half]
    hi_ref[...] = ne[:, half:]


def _edge(ea, xs, xd, w1a, b1, w1s, w1d, w2, b2, be=2000):
    e, de = ea.shape
    d = w1s.shape[0]
    dh = w2.shape[0]
    dn = w2.shape[1]
    half = dn // 2
    return pl.pallas_call(
        _edge_body,
        grid=(e // be,),
        in_specs=[
            pl.BlockSpec((be, de), lambda i: (i, 0)),
            pl.BlockSpec((be, d // 2), lambda i: (i, 0)),
            pl.BlockSpec((be, d // 2), lambda i: (i, 0)),
            pl.BlockSpec((de, dh), lambda i: (0, 0)),
            pl.BlockSpec((1, dh), lambda i: (0, 0)),
            pl.BlockSpec((d, dh), lambda i: (0, 0)),
            pl.BlockSpec((d, dh), lambda i: (0, 0)),
            pl.BlockSpec((dh, dn), lambda i: (0, 0)),
            pl.BlockSpec((1, dn), lambda i: (0, 0)),
        ],
        out_specs=[
            pl.BlockSpec((be, half), lambda i: (i, 0)),
            pl.BlockSpec((be, half), lambda i: (i, 0)),
        ],
        out_shape=[
            jax.ShapeDtypeStruct((e, half), jnp.float32),
            jax.ShapeDtypeStruct((e, half), jnp.float32),
        ],
    )(ea, xs, xd, w1a, b1, w1s, w1d, w2, b2)


def _make_scatter(es, n, h, s_count, c=80, rz=80):
    per_t = es // NS
    n_chunks = per_t // c
    n_pairs = n_chunks // 2
    nz = n // rz

    @functools.partial(
        pl.kernel,
        out_type=(
            jax.ShapeDtypeStruct((n, h), jnp.float32),
            jax.ShapeDtypeStruct((n, h), jnp.float32),
        ),
        mesh=_mesh(),
        scratch_types=[
            pltpu.VMEM((s_count * n_chunks, c), jnp.int32),
            pltpu.VMEM((2, c, h), jnp.float32),
            pltpu.VMEM((rz, h), jnp.float32),
            pltpu.VMEM_SHARED((n, h), jnp.float32),
        ] + [pltpu.SemaphoreType.DMA] * 4,
    )
    def k(*refs):
        los = refs[0:s_count]
        his = refs[s_count:2 * s_count]
        dst4_hbm = refs[2 * s_count]
        alo_hbm, ahi_hbm = refs[2 * s_count + 1], refs[2 * s_count + 2]
        idx_all, buf, zbuf, acc, sl0, sl1, sa0, sa1 = refs[2 * s_count + 3:]
        cid = lax.axis_index("c")
        tid = lax.axis_index("s")
        my_nz = (nz - tid + NS - 1) // NS

        def zrow(r, carry):
            for j in range(h // 16):
                zbuf[r, pl.ds(j * 16, 16)] = jnp.zeros((16,), jnp.float32)
            return carry

        lax.fori_loop(0, rz, zrow, 0)

        def zchunk(kk, carry):
            pltpu.sync_copy(zbuf, acc.at[pl.ds((tid + kk * NS) * rz, rz)])
            return carry

        lax.fori_loop(0, my_nz, zchunk, 0)
        for si in range(s_count):
            pltpu.sync_copy(
                dst4_hbm.at[si, tid],
                idx_all.at[pl.ds(si * n_chunks, n_chunks)])
        plsc.subcore_barrier()

        for ci, srcs in enumerate((los, his)):
            @pl.when(cid == ci)
            def _(srcs=srcs):
                for si in range(s_count):
                    src_ref = srcs[si]
                    ib = si * n_chunks

                    def ld(buf_s, sem, j, src_ref=src_ref):
                        pltpu.async_copy(
                            src_ref.at[pl.ds(tid * per_t + j * c, c)],
                            buf_s, sem)

                    def wld(buf_s, sem, src_ref=src_ref):
                        pltpu.make_async_copy(
                            src_ref.at[pl.ds(tid * per_t, c)],
                            buf_s, sem).wait()

                    def sc(buf_s, sem, j, ib=ib):
                        pltpu.async_copy(
                            buf_s, acc.at[idx_all.at[ib + j]], sem, add=True)

                    def wsc(buf_s, sem):
                        pltpu.make_async_copy(
                            buf_s, acc.at[idx_all.at[0]], sem).wait()

                    ld(buf.at[0], sl0, 0)

                    def body(i, carry):
                        j0 = 2 * i
                        @pl.when(i > 0)
                        def _():
                            wsc(buf.at[1], sa1)

                        ld(buf.at[1], sl1, j0 + 1)
                        wld(buf.at[0], sl0)
                        sc(buf.at[0], sa0, j0)
                        wsc(buf.at[0], sa0)
                        ld(buf.at[0], sl0, j0 + 2)
                        wld(buf.at[1], sl1)
                        sc(buf.at[1], sa1, j0 + 1)
                        return carry

                    lax.fori_loop(0, n_pairs, body, 0)
                    wld(buf.at[0], sl0)
                    sc(buf.at[0], sa0, n_chunks - 1)
                    wsc(buf.at[0], sa0)
                    wsc(buf.at[1], sa1)

        plsc.subcore_barrier()
        for ci, out_ref in enumerate((alo_hbm, ahi_hbm)):
            @pl.when(cid == ci)
            def _(out_ref=out_ref):
                def ochunk(kk, carry):
                    row = (tid + kk * NS) * rz
                    pltpu.sync_copy(acc.at[pl.ds(row, rz)],
                                    out_ref.at[pl.ds(row, rz)])
                    return carry

                lax.fori_loop(0, my_nz, ochunk, 0)

    return k


def _node_body(x_ref, al_ref, ah_ref, w3_ref, b3_ref, w4_ref, b4_ref, o_ref):
    n_in = jnp.concatenate([x_ref[...], al_ref[...], ah_ref[...]], axis=1)
    h2 = jnp.tanh(
        jnp.dot(n_in, w3_ref[...], preferred_element_type=jnp.float32)
        + b3_ref[...])
    o_ref[...] = jnp.tanh(
        jnp.dot(h2, w4_ref[...], preferred_element_type=jnp.float32)
        + b4_ref[...])


def _node(x, alo, ahi, w3, b3, w4, b4, bn=2000):
    n, d = x.shape
    half = alo.shape[1]
    dh = w3.shape[1]
    dn = w4.shape[1]
    return pl.pallas_call(
        _node_body,
        grid=(n // bn,),
        in_specs=[
            pl.BlockSpec((bn, d), lambda i: (i, 0)),
            pl.BlockSpec((bn, half), lambda i: (i, 0)),
            pl.BlockSpec((bn, half), lambda i: (i, 0)),
            pl.BlockSpec((d + 2 * half, dh), lambda i: (0, 0)),
            pl.BlockSpec((1, dh), lambda i: (0, 0)),
            pl.BlockSpec((dh, dn), lambda i: (0, 0)),
            pl.BlockSpec((1, dn), lambda i: (0, 0)),
        ],
        out_specs=pl.BlockSpec((bn, dn), lambda i: (i, 0)),
        out_shape=jax.ShapeDtypeStruct((n, dn), jnp.float32),
    )(x, alo, ahi, w3, b3, w4, b4)


def kernel(x, edge_attr, edge_index, W1, b1, W2, b2, W3, b3, W4, b4):
    n, d_node = x.shape
    e, d_edge = edge_attr.shape
    d_hid = W1.shape[1]
    src = edge_index[0].astype(jnp.int32)
    dst = edge_index[1].astype(jnp.int32)

    w1a = W1[:d_edge]
    w1s = W1[d_edge:d_edge + d_node]
    w1d = W1[d_edge + d_node:]

    xp = _packx(x)

    s_count = 1
    es = e // s_count
    gath = _make_gather(es, n, d_node)
    b1r, b2r = b1.reshape(1, -1), b2.reshape(1, -1)
    w1s_b = w1s.astype(jnp.bfloat16)
    w1d_b = w1d.astype(jnp.bfloat16)
    w2_b = W2.astype(jnp.bfloat16)
    ne_los, ne_his = [], []
    for s in range(s_count):
        sl_ = slice(s * es, (s + 1) * es)
        gxs, gxd = gath(xp, xp, src[sl_], dst[sl_])
        lo, hi = _edge(edge_attr[sl_], gxs, gxd,
                       w1a.astype(jnp.bfloat16), b1r,
                       w1s_b, w1d_b, w2_b, b2r)
        ne_los.append(lo)
        ne_his.append(hi)

    dst4 = dst.reshape(s_count, NS, -1, 80)
    alo, ahi = _make_scatter(es, n, d_node // 2, s_count)(
        *ne_los, *ne_his, dst4)
    return _node(
        x, alo, ahi, W3, b3.reshape(1, -1), W4, b4.reshape(1, -1))


# speedup vs baseline: 1.0077x; 1.0077x over previous
import functools

import jax
import jax.numpy as jnp
from jax import lax
from jax.experimental import pallas as pl
from jax.experimental.pallas import tpu as pltpu
from jax.experimental.pallas import tpu_sc as plsc

NC = 2
NS = 16
NW = NC * NS

_mesh = functools.partial(
    plsc.VectorSubcoreMesh,
    core_axis_name="c", subcore_axis_name="s", num_cores=NC, num_subcores=NS,
)


def _pack16(v):
    half = v.shape[1] // 2
    bits = jax.lax.bitcast_convert_type(v, jnp.uint32) + jnp.uint32(0x8000)
    lo = bits[:, :half] & jnp.uint32(0xFFFF0000)
    hi = bits[:, half:] >> jnp.uint32(16)
    return jax.lax.bitcast_convert_type(lo | hi, jnp.int32)


def _unpack16(w):
    u = jax.lax.bitcast_convert_type(w, jnp.uint32)
    lo = jax.lax.bitcast_convert_type(u & jnp.uint32(0xFFFF0000), jnp.float32)
    hi = jax.lax.bitcast_convert_type(u << jnp.uint32(16), jnp.float32)
    return lo, hi


def _packx_body(x_ref, o_ref):
    o_ref[...] = _pack16(x_ref[...])


def _packx(x, bn=2000):
    n, d = x.shape
    return pl.pallas_call(
        _packx_body,
        grid=(n // bn,),
        in_specs=[pl.BlockSpec((bn, d), lambda i: (i, 0))],
        out_specs=pl.BlockSpec((bn, d // 2), lambda i: (i, 0)),
        out_shape=jax.ShapeDtypeStruct((n, d // 2), jnp.int32),
    )(x)


def _make_gather(e, n, d, c=40):
    per_w = e // NW
    n_chunks = per_w // c
    assert n_chunks % 3 == 2 and n_chunks >= 5
    n_trips = (n_chunks - 2) // 3
    w = d // 2

    @functools.partial(
        pl.kernel,
        out_type=(
            jax.ShapeDtypeStruct((e, w), jnp.int32),
            jax.ShapeDtypeStruct((e, w), jnp.int32),
        ),
        mesh=_mesh(),
        scratch_types=[
            pltpu.VMEM((per_w,), jnp.int32),
            pltpu.VMEM((per_w,), jnp.int32),
            pltpu.VMEM((3, c, w), jnp.int32),
            pltpu.VMEM((3, c, w), jnp.int32),
            ([pltpu.SemaphoreType.DMA] * 3),
            ([pltpu.SemaphoreType.DMA] * 3),
            ([pltpu.SemaphoreType.DMA] * 3),
            ([pltpu.SemaphoreType.DMA] * 3),
        ],
    )
    def k(p_hbm, q_hbm, src_hbm, dst_hbm, gp_hbm, gq_hbm,
          idx_s, idx_d, bufp, bufq, sgp, swp, sgq, swq):
        wid = lax.axis_index("s") * NC + lax.axis_index("c")
        base = wid * per_w
        pltpu.sync_copy(src_hbm.at[pl.ds(base, per_w)], idx_s)
        pltpu.sync_copy(dst_hbm.at[pl.ds(base, per_w)], idx_d)

        def g(s, j):
            pltpu.async_copy(
                p_hbm.at[idx_s.at[pl.ds(j * c, c)]], bufp.at[s], sgp[s])
            pltpu.async_copy(
                q_hbm.at[idx_d.at[pl.ds(j * c, c)]], bufq.at[s], sgq[s])

        def wg(s):
            pltpu.make_async_copy(
                p_hbm.at[idx_s.at[pl.ds(0, c)]], bufp.at[s], sgp[s]).wait()
            pltpu.make_async_copy(
                q_hbm.at[idx_d.at[pl.ds(0, c)]], bufq.at[s], sgq[s]).wait()

        def w(s, j):
            pltpu.async_copy(
                bufp.at[s], gp_hbm.at[pl.ds(base + j * c, c)], swp[s])
            pltpu.async_copy(
                bufq.at[s], gq_hbm.at[pl.ds(base + j * c, c)], swq[s])

        def ww(s):
            pltpu.make_async_copy(
                bufp.at[s], gp_hbm.at[pl.ds(base, c)], swp[s]).wait()
            pltpu.make_async_copy(
                bufq.at[s], gq_hbm.at[pl.ds(base, c)], swq[s]).wait()

        g(0, 0)

        def body(i, carry):
            j0 = 3 * i
            for b in range(3):
                jb = j0 + b
                nxt = (b + 1) % 3
                if b < 2:
                    @pl.when(i > 0)
                    def _(nxt=nxt):
                        ww(nxt)
                else:
                    ww(nxt)
                g(nxt, jb + 1)
                wg(b)
                w(b, jb)
            return carry

        lax.fori_loop(0, n_trips, body, 0)
        ww(1)
        g(1, n_chunks - 1)
        wg(0)
        w(0, n_chunks - 2)
        wg(1)
        w(1, n_chunks - 1)
        ww(2)
        ww(0)
        ww(1)

    return k


def _edge_body(ea_ref, xs_ref, xd_ref, w1a_ref, b1_ref, w1s_ref, w1d_ref,
               w2_ref, b2_ref, lo_ref, hi_ref):
    ea = ea_ref[...]
    dn = w2_ref.shape[1]
    qw = w1s_ref.shape[0] // 2
    pre = (jnp.dot(ea, w1a_ref[...], preferred_element_type=jnp.float32)
           + b1_ref[...])
    for ref, wref in ((xs_ref, w1s_ref), (xd_ref, w1d_ref)):
        vlo, vhi = _unpack16(ref[...])
        pre = pre + jnp.dot(
            vlo.astype(jnp.bfloat16), wref[:qw, :],
            preferred_element_type=jnp.float32)
        pre = pre + jnp.dot(
            vhi.astype(jnp.bfloat16), wref[qw:, :],
            preferred_element_type=jnp.float32)
    h = jnp.tanh(pre).astype(jnp.bfloat16)
    ne = jnp.tanh(
        jnp.dot(h, w2_ref[...], preferred_element_type=jnp.float32)
        + b2_ref[...])
    half = dn // 2
    lo_ref[...] = ne[:, :half]
    hi_ref[...] = ne[:, half:]


def _edge(ea, xs, xd, w1a, b1, w1s, w1d, w2, b2, be=2000):
    e, de = ea.shape
    d = w1s.shape[0]
    dh = w2.shape[0]
    dn = w2.shape[1]
    half = dn // 2
    return pl.pallas_call(
        _edge_body,
        grid=(e // be,),
        in_specs=[
            pl.BlockSpec((be, de), lambda i: (i, 0)),
            pl.BlockSpec((be, d // 2), lambda i: (i, 0)),
            pl.BlockSpec((be, d // 2), lambda i: (i, 0)),
            pl.BlockSpec((de, dh), lambda i: (0, 0)),
            pl.BlockSpec((1, dh), lambda i: (0, 0)),
            pl.BlockSpec((d, dh), lambda i: (0, 0)),
            pl.BlockSpec((d, dh), lambda i: (0, 0)),
            pl.BlockSpec((dh, dn), lambda i: (0, 0)),
            pl.BlockSpec((1, dn), lambda i: (0, 0)),
        ],
        out_specs=[
            pl.BlockSpec((be, half), lambda i: (i, 0)),
            pl.BlockSpec((be, half), lambda i: (i, 0)),
        ],
        out_shape=[
            jax.ShapeDtypeStruct((e, half), jnp.float32),
            jax.ShapeDtypeStruct((e, half), jnp.float32),
        ],
    )(ea, xs, xd, w1a, b1, w1s, w1d, w2, b2)


def _make_scatter(es, n, h, s_count, c=80, rz=80):
    per_t = es // NS
    n_chunks = per_t // c
    n_pairs = n_chunks // 2
    nz = n // rz

    @functools.partial(
        pl.kernel,
        out_type=(
            jax.ShapeDtypeStruct((n, h), jnp.float32),
            jax.ShapeDtypeStruct((n, h), jnp.float32),
        ),
        mesh=_mesh(),
        scratch_types=[
            pltpu.VMEM((s_count * n_chunks, c), jnp.int32),
            pltpu.VMEM((2, c, h), jnp.float32),
            pltpu.VMEM((rz, h), jnp.float32),
            pltpu.VMEM_SHARED((n, h), jnp.float32),
        ] + [pltpu.SemaphoreType.DMA] * 4,
    )
    def k(*refs):
        los = refs[0:s_count]
        his = refs[s_count:2 * s_count]
        dst4_hbm = refs[2 * s_count]
        alo_hbm, ahi_hbm = refs[2 * s_count + 1], refs[2 * s_count + 2]
        idx_all, buf, zbuf, acc, sl0, sl1, sa0, sa1 = refs[2 * s_count + 3:]
        cid = lax.axis_index("c")
        tid = lax.axis_index("s")
        my_nz = (nz - tid + NS - 1) // NS

        def zrow(r, carry):
            for j in range(h // 16):
                zbuf[r, pl.ds(j * 16, 16)] = jnp.zeros((16,), jnp.float32)
            return carry

        lax.fori_loop(0, rz, zrow, 0)

        def zchunk(kk, carry):
            pltpu.sync_copy(zbuf, acc.at[pl.ds((tid + kk * NS) * rz, rz)])
            return carry

        lax.fori_loop(0, my_nz, zchunk, 0)
        for si in range(s_count):
            pltpu.sync_copy(
                dst4_hbm.at[si, tid],
                idx_all.at[pl.ds(si * n_chunks, n_chunks)])
        plsc.subcore_barrier()

        for ci, srcs in enumerate((los, his)):
            @pl.when(cid == ci)
            def _(srcs=srcs):
                for si in range(s_count):
                    src_ref = srcs[si]
                    ib = si * n_chunks

                    def ld(buf_s, sem, j, src_ref=src_ref):
                        pltpu.async_copy(
                            src_ref.at[pl.ds(tid * per_t + j * c, c)],
                            buf_s, sem)

                    def wld(buf_s, sem, src_ref=src_ref):
                        pltpu.make_async_copy(
                            src_ref.at[pl.ds(tid * per_t, c)],
                            buf_s, sem).wait()

                    def sc(buf_s, sem, j, ib=ib):
                        pltpu.async_copy(
                            buf_s, acc.at[idx_all.at[ib + j]], sem, add=True)

                    def wsc(buf_s, sem):
                        pltpu.make_async_copy(
                            buf_s, acc.at[idx_all.at[0]], sem).wait()

                    ld(buf.at[0], sl0, 0)

                    def body(i, carry):
                        j0 = 2 * i
                        @pl.when(i > 0)
                        def _():
                            wsc(buf.at[1], sa1)

                        ld(buf.at[1], sl1, j0 + 1)
                        wld(buf.at[0], sl0)
                        sc(buf.at[0], sa0, j0)
                        wsc(buf.at[0], sa0)
                        ld(buf.at[0], sl0, j0 + 2)
                        wld(buf.at[1], sl1)
                        sc(buf.at[1], sa1, j0 + 1)
                        return carry

                    lax.fori_loop(0, n_pairs, body, 0)
                    wld(buf.at[0], sl0)
                    sc(buf.at[0], sa0, n_chunks - 1)
                    wsc(buf.at[0], sa0)
                    wsc(buf.at[1], sa1)

        plsc.subcore_barrier()
        for ci, out_ref in enumerate((alo_hbm, ahi_hbm)):
            @pl.when(cid == ci)
            def _(out_ref=out_ref):
                def ochunk(kk, carry):
                    row = (tid + kk * NS) * rz
                    pltpu.sync_copy(acc.at[pl.ds(row, rz)],
                                    out_ref.at[pl.ds(row, rz)])
                    return carry

                lax.fori_loop(0, my_nz, ochunk, 0)

    return k


def _node_body(x_ref, al_ref, ah_ref, w3_ref, b3_ref, w4_ref, b4_ref, o_ref):
    n_in = jnp.concatenate([x_ref[...], al_ref[...], ah_ref[...]], axis=1)
    h2 = jnp.tanh(
        jnp.dot(n_in, w3_ref[...], preferred_element_type=jnp.float32)
        + b3_ref[...])
    o_ref[...] = jnp.tanh(
        jnp.dot(h2, w4_ref[...], preferred_element_type=jnp.float32)
        + b4_ref[...])


def _node(x, alo, ahi, w3, b3, w4, b4, bn=2000):
    n, d = x.shape
    half = alo.shape[1]
    dh = w3.shape[1]
    dn = w4.shape[1]
    return pl.pallas_call(
        _node_body,
        grid=(n // bn,),
        in_specs=[
            pl.BlockSpec((bn, d), lambda i: (i, 0)),
            pl.BlockSpec((bn, half), lambda i: (i, 0)),
            pl.BlockSpec((bn, half), lambda i: (i, 0)),
            pl.BlockSpec((d + 2 * half, dh), lambda i: (0, 0)),
            pl.BlockSpec((1, dh), lambda i: (0, 0)),
            pl.BlockSpec((dh, dn), lambda i: (0, 0)),
            pl.BlockSpec((1, dn), lambda i: (0, 0)),
        ],
        out_specs=pl.BlockSpec((bn, dn), lambda i: (i, 0)),
        out_shape=jax.ShapeDtypeStruct((n, dn), jnp.float32),
    )(x, alo, ahi, w3, b3, w4, b4)


def kernel(x, edge_attr, edge_index, W1, b1, W2, b2, W3, b3, W4, b4):
    n, d_node = x.shape
    e, d_edge = edge_attr.shape
    d_hid = W1.shape[1]
    src = edge_index[0].astype(jnp.int32)
    dst = edge_index[1].astype(jnp.int32)

    w1a = W1[:d_edge]
    w1s = W1[d_edge:d_edge + d_node]
    w1d = W1[d_edge + d_node:]

    xp = _packx(x)

    s_count = 1
    es = e // s_count
    gath = _make_gather(es, n, d_node)
    b1r, b2r = b1.reshape(1, -1), b2.reshape(1, -1)
    w1s_b = w1s.astype(jnp.bfloat16)
    w1d_b = w1d.astype(jnp.bfloat16)
    w2_b = W2.astype(jnp.bfloat16)
    ne_los, ne_his = [], []
    for s in range(s_count):
        sl_ = slice(s * es, (s + 1) * es)
        gxs, gxd = gath(xp, xp, src[sl_], dst[sl_])
        lo, hi = _edge(edge_attr[sl_], gxs, gxd,
                       w1a, b1r, w1s_b, w1d_b, w2_b, b2r)
        ne_los.append(lo)
        ne_his.append(hi)

    dst4 = dst.reshape(s_count, NS, -1, 80)
    alo, ahi = _make_scatter(es, n, d_node // 2, s_count)(
        *ne_los, *ne_his, dst4)
    return _node(
        x, alo, ahi, W3, b3.reshape(1, -1), W4, b4.reshape(1, -1))
